# baseline (speedup 1.0000x reference)
import functools

import jax
import jax.numpy as jnp
from jax import lax
from jax.experimental import pallas as pl
from jax.experimental.pallas import tpu as pltpu
from jax.experimental.pallas import tpu_sc as plsc

NC = 2
NS = 16
NW = NC * NS
BB = 128
L = 16


def _gather_rows(ids_t, table):
    s, b = ids_t.shape
    v, d = table.shape
    n_bb = b // BB
    assert n_bb == NW
    mesh = plsc.VectorSubcoreMesh(core_axis_name="c", subcore_axis_name="s")

    @functools.partial(
        pl.kernel,
        out_type=jax.ShapeDtypeStruct((s, d // 8, n_bb, 8, BB), jnp.float32),
        mesh=mesh,
        scratch_types=[
            pltpu.VMEM((s, BB), jnp.int32),
            pltpu.VMEM((2, BB, d), jnp.float32),
            pltpu.VMEM((2, d // 8, 8, BB), jnp.float32),
            pltpu.SemaphoreType.DMA,
            [pltpu.SemaphoreType.DMA] * 2,
        ],
        compiler_params=pltpu.CompilerParams(
            use_tc_tiling_on_sc=False, needs_layout_passes=False
        ),
    )
    def body(ids_hbm, table_hbm, out_hbm, idx_v, grow, tbuf, gsem, osems):
        wid = lax.axis_index("s") * NC + lax.axis_index("c")
        pltpu.sync_copy(ids_hbm.at[:, pl.ds(wid * BB, BB)], idx_v)

        def issue_gather(si, k):
            pltpu.async_copy(table_hbm.at[idx_v.at[si]], grow.at[k], gsem)

        issue_gather(0, 0)
        row_iota = [lax.iota(jnp.int32, L) + g * L for g in range(BB // L)]

        def step(p, carry):
            for k in range(2):
                si = p * 2 + k

                @pl.when(si + 1 < s)
                def _():
                    issue_gather(si + 1, 1 - k)

                pltpu.make_async_copy(
                    table_hbm.at[idx_v.at[0]], grow.at[k], gsem
                ).wait()

                @pl.when(si >= 2)
                def _():
                    pltpu.make_async_copy(
                        tbuf.at[k], out_hbm.at[0, :, 0], osems[k]
                    ).wait()

                for dd in range(d):
                    col = jnp.full((L,), dd, jnp.int32)
                    for g in range(BB // L):
                        vec = plsc.load_gather(grow.at[k], [row_iota[g], col])
                        tbuf[k, dd // 8, dd % 8, pl.ds(g * L, L)] = vec

                pltpu.async_copy(tbuf.at[k], out_hbm.at[si, :, wid], osems[k])
            return carry

        lax.fori_loop(0, s // 2, step, 0)
        for k in range(2):
            pltpu.make_async_copy(
                tbuf.at[k], out_hbm.at[0, :, 0], osems[k]
            ).wait()

    return body(ids_t, table)


_gather_jit = jax.jit(_gather_rows)


def kernel(ids, table):
    b, s = ids.shape
    _, d = table.shape
    out5 = _gather_jit(ids.T, table)
    return out5.transpose(2, 4, 0, 1, 3).reshape(b, s, d)
